# baseline (speedup 1.0000x reference)
import functools

import numpy as np

import jax
import jax.numpy as jnp
from jax import lax
from jax.experimental import pallas as pl
from jax.experimental.pallas import tpu as pltpu
from jax.experimental.pallas import tpu_sc as plsc

VOCAB = 1000000
D_MODEL = 32
BATCH = 16384
HIST = 50
B_FLAT = BATCH * HIST

_NC = 2
_NS = 16
_NW = _NC * _NS

_mesh = plsc.VectorSubcoreMesh(core_axis_name="c", subcore_axis_name="s")

_A_CH = 256
_A_FULL = 3906
_A_ITERS = 124


@functools.partial(
    pl.kernel,
    mesh=_mesh,
    compiler_params=pltpu.CompilerParams(needs_layout_passes=False),
    out_type=jax.ShapeDtypeStruct((VOCAB * D_MODEL,), jnp.float32),
    scratch_types=[
        tuple(pltpu.VMEM((D_MODEL, _A_CH), jnp.float32) for _ in range(2)),
        tuple(pltpu.VMEM((_A_CH * D_MODEL,), jnp.float32) for _ in range(2)),
        tuple(pltpu.SemaphoreType.DMA for _ in range(2)),
        tuple(pltpu.SemaphoreType.DMA for _ in range(2)),
    ],
)
def _detranspose(tbl_t, tail_rows, out_hbm, vins, vouts, isems, osems):
    wid = lax.axis_index("s") * _NC + lax.axis_index("c")
    lanes16 = lax.iota(jnp.int32, 16)
    scat_idx = [lanes16 * D_MODEL + d for d in range(D_MODEL)]

    def col0(k):
        return (wid + k * _NW) * _A_CH

    def start_in(k, b):
        @pl.when(wid + k * _NW < _A_FULL)
        def _():
            pltpu.async_copy(
                tbl_t.at[:, pl.ds(col0(k), _A_CH)], vins[b], isems[b]
            )

    def wait_in(k, b):
        @pl.when(wid + k * _NW < _A_FULL)
        def _():
            pltpu.make_async_copy(
                tbl_t.at[:, pl.ds(col0(k), _A_CH)], vins[b], isems[b]
            ).wait()

    def start_out(k, b):
        @pl.when(wid + k * _NW < _A_FULL)
        def _():
            pltpu.async_copy(
                vouts[b], out_hbm.at[pl.ds(col0(k) * D_MODEL, _A_CH * D_MODEL)],
                osems[b],
            )

    def wait_out(k, b):
        @pl.when(wid + k * _NW < _A_FULL)
        def _():
            pltpu.make_async_copy(
                vouts[b], out_hbm.at[pl.ds(col0(k) * D_MODEL, _A_CH * D_MODEL)],
                osems[b],
            ).wait()

    def compute(k, b):
        @pl.when(wid + k * _NW < _A_FULL)
        def _():
            vin = vins[b]
            vout = vouts[b]
            for il0 in range(0, _A_CH, 16):
                dst = vout.at[pl.ds(il0 * D_MODEL, 16 * D_MODEL)]
                for d in range(D_MODEL):
                    v = vin[d, pl.ds(il0, 16)]
                    plsc.store_scatter(dst, [scat_idx[d]], v)

    start_in(0, 0)
    start_in(1, 1)

    def chunk_loop(k2, carry):
        for b in range(2):
            k = k2 * 2 + b
            wait_in(k, b)

            @pl.when(k >= 2)
            def _():
                wait_out(k - 2, b)

            compute(k, b)
            start_out(k, b)
            start_in(k + 2, b)
        return carry

    lax.fori_loop(0, _A_ITERS // 2, chunk_loop, 0)
    wait_out(_A_ITERS - 2, 0)
    wait_out(_A_ITERS - 1, 1)

    @pl.when(wid == 0)
    def _():
        pltpu.sync_copy(tail_rows, vouts[0].at[pl.ds(0, 64 * D_MODEL)])
        pltpu.sync_copy(
            vouts[0].at[pl.ds(0, 64 * D_MODEL)],
            out_hbm.at[pl.ds(999936 * D_MODEL, 64 * D_MODEL)],
        )


_B_CH = 1024
_B_PER_W = B_FLAT // _NW
_B_ITERS = _B_PER_W // _B_CH


@functools.partial(
    pl.kernel,
    mesh=_mesh,
    compiler_params=pltpu.CompilerParams(
        use_tc_tiling_on_sc=False, needs_layout_passes=False
    ),
    out_type=jax.ShapeDtypeStruct((B_FLAT, D_MODEL), jnp.float32),
    scratch_types=[
        pltpu.VMEM((_B_CH,), jnp.int32),
        pltpu.VMEM((_B_CH, D_MODEL), jnp.float32),
        tuple(pltpu.VMEM((128,), jnp.int32) for _ in range(8)),
        pltpu.SemaphoreType.DMA,
    ],
)
def _gather_scatter(idx_hbm, tbl_lin, out_hbm, idxv, rows, drefs, sem):
    wid = lax.axis_index("s") * _NC + lax.axis_index("c")
    base = wid * _B_PER_W
    rows16 = lax.iota(jnp.int32, 16)

    def chunk(kc, carry):
        j0 = base + kc * _B_CH
        pltpu.sync_copy(idx_hbm.at[pl.ds(j0, _B_CH)], idxv)
        pltpu.async_copy(tbl_lin.at[idxv], rows, sem).wait()
        for sub in range(8):
            dref = drefs[sub]
            for l in range(8):
                jv = jnp.full((16,), j0 + sub * 128 + l * 16, jnp.int32) + rows16
                h = jv % HIST
                b = jv // HIST
                dref[pl.ds(l * 16, 16)] = h * BATCH + b
            pltpu.async_copy(
                rows.at[pl.ds(sub * 128, 128)], out_hbm.at[dref], sem
            ).wait()
        return carry

    lax.fori_loop(0, _B_ITERS, chunk, 0)


_C_UNITS = (HIST * BATCH) // (128 * _NW)


@functools.partial(
    pl.kernel,
    mesh=_mesh,
    compiler_params=pltpu.CompilerParams(needs_layout_passes=False),
    out_type=jax.ShapeDtypeStruct((HIST, D_MODEL, BATCH), jnp.float32),
    scratch_types=[
        tuple(pltpu.VMEM((128 * D_MODEL,), jnp.float32) for _ in range(2)),
        tuple(pltpu.VMEM((D_MODEL, 128), jnp.float32) for _ in range(2)),
        tuple(pltpu.SemaphoreType.DMA for _ in range(2)),
        tuple(pltpu.SemaphoreType.DMA for _ in range(2)),
    ],
)
def _assemble(flat_in, out_hbm, vins, vtiles, isems, osems):
    wid = lax.axis_index("s") * _NC + lax.axis_index("c")
    lanes16 = lax.iota(jnp.int32, 16)
    gat_idx = [
        [lanes16 * D_MODEL + (l * 16 * D_MODEL + d) for l in range(8)]
        for d in range(D_MODEL)
    ]

    def src_slice(u):
        uu = wid + u * _NW
        h = uu // 128
        b0 = (uu % 128) * 128
        return flat_in.at[pl.ds((h * BATCH + b0) * D_MODEL, 128 * D_MODEL)]

    def dst_slice(u):
        uu = wid + u * _NW
        h = uu // 128
        b0 = (uu % 128) * 128
        return out_hbm.at[h, :, pl.ds(b0, 128)]

    def start_in(u, b):
        @pl.when(u < _C_UNITS)
        def _():
            pltpu.async_copy(src_slice(u), vins[b], isems[b])

    def unit(u2, carry):
        for b in range(2):
            u = u2 * 2 + b
            pltpu.make_async_copy(src_slice(u), vins[b], isems[b]).wait()

            @pl.when(u >= 2)
            def _():
                pltpu.make_async_copy(
                    vtiles[b], dst_slice(u - 2), osems[b]
                ).wait()

            vin = vins[b]
            vtile = vtiles[b]
            for d in range(D_MODEL):
                for l in range(8):
                    vtile[d, pl.ds(l * 16, 16)] = plsc.load_gather(
                        vin, [gat_idx[d][l]]
                    )

            pltpu.async_copy(vtile, dst_slice(u), osems[b])
            start_in(u + 2, b)
        return carry

    start_in(0, 0)
    start_in(1, 1)
    lax.fori_loop(0, _C_UNITS // 2, unit, 0)
    pltpu.make_async_copy(vtiles[0], dst_slice(_C_UNITS - 2), osems[0]).wait()
    pltpu.make_async_copy(vtiles[1], dst_slice(_C_UNITS - 1), osems[1]).wait()


def kernel(inputs, embedding_matrix):
    tbl_t = jnp.swapaxes(embedding_matrix, 0, 1)
    flat_idx = inputs.reshape(B_FLAT).astype(jnp.int32)
    tail_rows = lax.slice(
        embedding_matrix, (999936, 0), (VOCAB, D_MODEL)
    ).reshape(64 * D_MODEL)
    tbl_lin = _detranspose(tbl_t, tail_rows).reshape(VOCAB, D_MODEL)
    g = _gather_scatter(flat_idx, tbl_lin)
    out_t = _assemble(g.reshape(B_FLAT * D_MODEL))
    return jnp.transpose(out_t, (2, 0, 1))


# speedup vs baseline: 1.6220x; 1.6220x over previous
import functools

import numpy as np

import jax
import jax.numpy as jnp
from jax import lax
from jax.experimental import pallas as pl
from jax.experimental.pallas import tpu as pltpu
from jax.experimental.pallas import tpu_sc as plsc

VOCAB = 1000000
D_MODEL = 32
BATCH = 16384
HIST = 50
B_FLAT = BATCH * HIST

_NC = 2
_NS = 16
_NW = _NC * _NS

_mesh = plsc.VectorSubcoreMesh(core_axis_name="c", subcore_axis_name="s")

_A_CH = 512
_A_FULL = 1953
_A_ITERS = 62


@functools.partial(
    pl.kernel,
    mesh=_mesh,
    compiler_params=pltpu.CompilerParams(needs_layout_passes=False),
    out_type=jax.ShapeDtypeStruct((VOCAB * D_MODEL,), jnp.float32),
    scratch_types=[
        tuple(pltpu.VMEM((D_MODEL, _A_CH), jnp.float32) for _ in range(2)),
        tuple(pltpu.VMEM((_A_CH * D_MODEL,), jnp.float32) for _ in range(2)),
        tuple(pltpu.SemaphoreType.DMA for _ in range(2)),
        tuple(pltpu.SemaphoreType.DMA for _ in range(2)),
    ],
)
def _detranspose(tbl_t, tail_rows, out_hbm, vins, vouts, isems, osems):
    wid = lax.axis_index("s") * _NC + lax.axis_index("c")
    rows16 = lax.iota(jnp.int32, 16)

    def col0(k):
        return (wid + k * _NW) * _A_CH

    def start_in(k, b):
        @pl.when(wid + k * _NW < _A_FULL)
        def _():
            pltpu.async_copy(
                tbl_t.at[:, pl.ds(col0(k), _A_CH)], vins[b], isems[b]
            )

    def wait_in(k, b):
        @pl.when(wid + k * _NW < _A_FULL)
        def _():
            pltpu.make_async_copy(
                tbl_t.at[:, pl.ds(col0(k), _A_CH)], vins[b], isems[b]
            ).wait()

    def start_out(k, b):
        @pl.when(wid + k * _NW < _A_FULL)
        def _():
            pltpu.async_copy(
                vouts[b], out_hbm.at[pl.ds(col0(k) * D_MODEL, _A_CH * D_MODEL)],
                osems[b],
            )

    def wait_out(k, b):
        @pl.when(wid + k * _NW < _A_FULL)
        def _():
            pltpu.make_async_copy(
                vouts[b], out_hbm.at[pl.ds(col0(k) * D_MODEL, _A_CH * D_MODEL)],
                osems[b],
            ).wait()

    def compute(k, b):
        @pl.when(wid + k * _NW < _A_FULL)
        def _():
            vin = vins[b]
            vout = vouts[b]

            @plsc.parallel_loop(0, _A_CH, unroll=8)
            def _(il):
                cols = jnp.full((16,), il, jnp.int32)
                vout[pl.ds(il * D_MODEL, 16)] = plsc.load_gather(
                    vin, [rows16, cols]
                )
                vout[pl.ds(il * D_MODEL + 16, 16)] = plsc.load_gather(
                    vin, [rows16 + 16, cols]
                )

    start_in(0, 0)
    start_in(1, 1)

    def chunk_loop(k2, carry):
        for b in range(2):
            k = k2 * 2 + b
            wait_in(k, b)

            @pl.when(k >= 2)
            def _():
                wait_out(k - 2, b)

            compute(k, b)
            start_out(k, b)
            start_in(k + 2, b)
        return carry

    lax.fori_loop(0, _A_ITERS // 2, chunk_loop, 0)
    wait_out(_A_ITERS - 2, 0)
    wait_out(_A_ITERS - 1, 1)

    @pl.when(wid == 0)
    def _():
        pltpu.sync_copy(tail_rows, vouts[0].at[pl.ds(0, 64 * D_MODEL)])
        pltpu.sync_copy(
            vouts[0].at[pl.ds(0, 64 * D_MODEL)],
            out_hbm.at[pl.ds(999936 * D_MODEL, 64 * D_MODEL)],
        )


_B_CH = 1024
_B_PER_W = B_FLAT // _NW
_B_ITERS = _B_PER_W // _B_CH


@functools.partial(
    pl.kernel,
    mesh=_mesh,
    compiler_params=pltpu.CompilerParams(
        use_tc_tiling_on_sc=False, needs_layout_passes=False
    ),
    out_type=jax.ShapeDtypeStruct((B_FLAT, D_MODEL), jnp.float32),
    scratch_types=[
        pltpu.VMEM((_B_CH,), jnp.int32),
        pltpu.VMEM((_B_CH, D_MODEL), jnp.float32),
        tuple(pltpu.VMEM((128,), jnp.int32) for _ in range(8)),
        pltpu.SemaphoreType.DMA,
    ],
)
def _gather_scatter(idx_hbm, tbl_lin, out_hbm, idxv, rows, drefs, sem):
    wid = lax.axis_index("s") * _NC + lax.axis_index("c")
    base = wid * _B_PER_W
    rows16 = lax.iota(jnp.int32, 16)

    def chunk(kc, carry):
        j0 = base + kc * _B_CH
        pltpu.sync_copy(idx_hbm.at[pl.ds(j0, _B_CH)], idxv)
        pltpu.async_copy(tbl_lin.at[idxv], rows, sem).wait()
        for sub in range(8):
            dref = drefs[sub]
            for l in range(8):
                jv = jnp.full((16,), j0 + sub * 128 + l * 16, jnp.int32) + rows16
                h = jv % HIST
                b = jv // HIST
                dref[pl.ds(l * 16, 16)] = h * BATCH + b
            pltpu.async_copy(
                rows.at[pl.ds(sub * 128, 128)], out_hbm.at[dref], sem
            ).wait()
        return carry

    lax.fori_loop(0, _B_ITERS, chunk, 0)


_C_UNITS = (HIST * BATCH) // (128 * _NW)


@functools.partial(
    pl.kernel,
    mesh=_mesh,
    compiler_params=pltpu.CompilerParams(needs_layout_passes=False),
    out_type=jax.ShapeDtypeStruct((HIST, D_MODEL, BATCH), jnp.float32),
    scratch_types=[
        tuple(pltpu.VMEM((128 * D_MODEL,), jnp.float32) for _ in range(2)),
        tuple(pltpu.VMEM((D_MODEL, 128), jnp.float32) for _ in range(2)),
        tuple(pltpu.SemaphoreType.DMA for _ in range(2)),
        tuple(pltpu.SemaphoreType.DMA for _ in range(2)),
    ],
)
def _assemble(flat_in, out_hbm, vins, vtiles, isems, osems):
    wid = lax.axis_index("s") * _NC + lax.axis_index("c")
    lanes = lax.iota(jnp.int32, 16) * D_MODEL

    def src_slice(u):
        uu = wid + u * _NW
        h = uu // 128
        b0 = (uu % 128) * 128
        return flat_in.at[pl.ds((h * BATCH + b0) * D_MODEL, 128 * D_MODEL)]

    def dst_slice(u):
        uu = wid + u * _NW
        h = uu // 128
        b0 = (uu % 128) * 128
        return out_hbm.at[h, :, pl.ds(b0, 128)]

    def start_in(u, b):
        @pl.when(u < _C_UNITS)
        def _():
            pltpu.async_copy(src_slice(u), vins[b], isems[b])

    def unit(u2, carry):
        for b in range(2):
            u = u2 * 2 + b
            pltpu.make_async_copy(src_slice(u), vins[b], isems[b]).wait()

            @pl.when(u >= 2)
            def _():
                pltpu.make_async_copy(
                    vtiles[b], dst_slice(u - 2), osems[b]
                ).wait()

            vin = vins[b]
            vtile = vtiles[b]

            @plsc.parallel_loop(0, 256, unroll=8)
            def _(p):
                d = p % D_MODEL
                l = p // D_MODEL
                v = plsc.load_gather(vin, [lanes + (l * 16 * D_MODEL + d)])
                vtile[d, pl.ds(l * 16, 16)] = v

            pltpu.async_copy(vtile, dst_slice(u), osems[b])
            start_in(u + 2, b)
        return carry

    start_in(0, 0)
    start_in(1, 1)
    lax.fori_loop(0, _C_UNITS // 2, unit, 0)
    pltpu.make_async_copy(vtiles[0], dst_slice(_C_UNITS - 2), osems[0]).wait()
    pltpu.make_async_copy(vtiles[1], dst_slice(_C_UNITS - 1), osems[1]).wait()


def kernel(inputs, embedding_matrix):
    tbl_t = jnp.swapaxes(embedding_matrix, 0, 1)
    flat_idx = inputs.reshape(B_FLAT).astype(jnp.int32)
    tail_rows = lax.slice(
        embedding_matrix, (999936, 0), (VOCAB, D_MODEL)
    ).reshape(64 * D_MODEL)
    tbl_lin = _detranspose(tbl_t, tail_rows).reshape(VOCAB, D_MODEL)
    g = _gather_scatter(flat_idx, tbl_lin)
    out_t = _assemble(g.reshape(B_FLAT * D_MODEL))
    return jnp.transpose(out_t, (2, 0, 1))


# speedup vs baseline: 1.7246x; 1.0632x over previous
import functools

import numpy as np

import jax
import jax.numpy as jnp
from jax import lax
from jax.experimental import pallas as pl
from jax.experimental.pallas import tpu as pltpu
from jax.experimental.pallas import tpu_sc as plsc

VOCAB = 1000000
D_MODEL = 32
BATCH = 16384
HIST = 50
B_FLAT = BATCH * HIST

_NC = 2
_NS = 16
_NW = _NC * _NS

_mesh = plsc.VectorSubcoreMesh(core_axis_name="c", subcore_axis_name="s")

_A_CH = 512
_A_FULL = 1953
_A_ITERS = 62


@functools.partial(
    pl.kernel,
    mesh=_mesh,
    compiler_params=pltpu.CompilerParams(needs_layout_passes=False),
    out_type=jax.ShapeDtypeStruct((VOCAB * D_MODEL,), jnp.float32),
    scratch_types=[
        tuple(pltpu.VMEM((D_MODEL, _A_CH), jnp.float32) for _ in range(2)),
        tuple(pltpu.VMEM((_A_CH * D_MODEL,), jnp.float32) for _ in range(2)),
        tuple(pltpu.SemaphoreType.DMA for _ in range(2)),
        tuple(pltpu.SemaphoreType.DMA for _ in range(2)),
    ],
)
def _detranspose(tbl_t, tail_rows, out_hbm, vins, vouts, isems, osems):
    wid = lax.axis_index("s") * _NC + lax.axis_index("c")
    rows16 = lax.iota(jnp.int32, 16)

    def col0(k):
        return (wid + k * _NW) * _A_CH

    def start_in(k, b):
        @pl.when(wid + k * _NW < _A_FULL)
        def _():
            pltpu.async_copy(
                tbl_t.at[:, pl.ds(col0(k), _A_CH)], vins[b], isems[b]
            )

    def wait_in(k, b):
        @pl.when(wid + k * _NW < _A_FULL)
        def _():
            pltpu.make_async_copy(
                tbl_t.at[:, pl.ds(col0(k), _A_CH)], vins[b], isems[b]
            ).wait()

    def start_out(k, b):
        @pl.when(wid + k * _NW < _A_FULL)
        def _():
            pltpu.async_copy(
                vouts[b], out_hbm.at[pl.ds(col0(k) * D_MODEL, _A_CH * D_MODEL)],
                osems[b],
            )

    def wait_out(k, b):
        @pl.when(wid + k * _NW < _A_FULL)
        def _():
            pltpu.make_async_copy(
                vouts[b], out_hbm.at[pl.ds(col0(k) * D_MODEL, _A_CH * D_MODEL)],
                osems[b],
            ).wait()

    def compute(k, b):
        @pl.when(wid + k * _NW < _A_FULL)
        def _():
            vin = vins[b]
            vout = vouts[b]

            @plsc.parallel_loop(0, _A_CH, unroll=16)
            def _(il):
                cols = jnp.full((16,), il, jnp.int32)
                vout[pl.ds(il * D_MODEL, 16)] = plsc.load_gather(
                    vin, [rows16, cols]
                )
                vout[pl.ds(il * D_MODEL + 16, 16)] = plsc.load_gather(
                    vin, [rows16 + 16, cols]
                )

    start_in(0, 0)
    start_in(1, 1)

    def chunk_loop(k2, carry):
        for b in range(2):
            k = k2 * 2 + b
            wait_in(k, b)

            @pl.when(k >= 2)
            def _():
                wait_out(k - 2, b)

            compute(k, b)
            start_out(k, b)
            start_in(k + 2, b)
        return carry

    lax.fori_loop(0, _A_ITERS // 2, chunk_loop, 0)
    wait_out(_A_ITERS - 2, 0)
    wait_out(_A_ITERS - 1, 1)

    @pl.when(wid == 0)
    def _():
        pltpu.sync_copy(tail_rows, vouts[0].at[pl.ds(0, 64 * D_MODEL)])
        pltpu.sync_copy(
            vouts[0].at[pl.ds(0, 64 * D_MODEL)],
            out_hbm.at[pl.ds(999936 * D_MODEL, 64 * D_MODEL)],
        )


_B_CH = 1024
_B_PER_W = B_FLAT // _NW
_B_ITERS = _B_PER_W // _B_CH


@functools.partial(
    pl.kernel,
    mesh=_mesh,
    compiler_params=pltpu.CompilerParams(
        use_tc_tiling_on_sc=False, needs_layout_passes=False
    ),
    out_type=jax.ShapeDtypeStruct((B_FLAT, D_MODEL), jnp.float32),
    scratch_types=[
        tuple(pltpu.VMEM((_B_CH,), jnp.int32) for _ in range(2)),
        tuple(pltpu.VMEM((_B_CH, D_MODEL), jnp.float32) for _ in range(2)),
        tuple(pltpu.VMEM((128,), jnp.int32) for _ in range(8)),
        tuple(pltpu.SemaphoreType.DMA for _ in range(2)),
        tuple(pltpu.SemaphoreType.DMA for _ in range(2)),
    ],
)
def _gather_scatter(idx_hbm, tbl_lin, out_hbm, idxvs, rowss, drefs, gsems, ssems):
    wid = lax.axis_index("s") * _NC + lax.axis_index("c")
    base = wid * _B_PER_W
    rows16 = lax.iota(jnp.int32, 16)

    def start_gather(kc, b):
        @pl.when(kc < _B_ITERS)
        def _():
            j0 = base + kc * _B_CH
            pltpu.sync_copy(idx_hbm.at[pl.ds(j0, _B_CH)], idxvs[b])
            pltpu.async_copy(tbl_lin.at[idxvs[b]], rowss[b], gsems[b])

    start_gather(0, 0)
    start_gather(1, 1)

    def chunk(k2, carry):
        for b in range(2):
            kc = k2 * 2 + b

            @pl.when(kc < _B_ITERS)
            def _():
                j0 = base + kc * _B_CH
                rows = rowss[b]
                pltpu.make_async_copy(
                    tbl_lin.at[idxvs[b]], rows, gsems[b]
                ).wait()
                for sub in range(8):
                    dref = drefs[sub]
                    for l in range(8):
                        jv = (
                            jnp.full(
                                (16,), j0 + sub * 128 + l * 16, jnp.int32
                            )
                            + rows16
                        )
                        h = jv % HIST
                        bb = jv // HIST
                        dref[pl.ds(l * 16, 16)] = h * BATCH + bb
                    pltpu.async_copy(
                        rows.at[pl.ds(sub * 128, 128)],
                        out_hbm.at[dref],
                        ssems[b],
                    )
                for sub in range(8):
                    pltpu.make_async_copy(
                        rows.at[pl.ds(sub * 128, 128)],
                        out_hbm.at[drefs[sub]],
                        ssems[b],
                    ).wait()

            start_gather(kc + 2, b)
        return carry

    lax.fori_loop(0, (_B_ITERS + 2) // 2, chunk, 0)


_C_UNITS = (HIST * BATCH) // (128 * _NW)


@functools.partial(
    pl.kernel,
    mesh=_mesh,
    compiler_params=pltpu.CompilerParams(needs_layout_passes=False),
    out_type=jax.ShapeDtypeStruct((HIST, D_MODEL, BATCH), jnp.float32),
    scratch_types=[
        tuple(pltpu.VMEM((128 * D_MODEL,), jnp.float32) for _ in range(2)),
        tuple(pltpu.VMEM((D_MODEL, 128), jnp.float32) for _ in range(2)),
        tuple(pltpu.SemaphoreType.DMA for _ in range(2)),
        tuple(pltpu.SemaphoreType.DMA for _ in range(2)),
    ],
)
def _assemble(flat_in, out_hbm, vins, vtiles, isems, osems):
    wid = lax.axis_index("s") * _NC + lax.axis_index("c")
    lanes = lax.iota(jnp.int32, 16) * D_MODEL

    def src_slice(u):
        uu = wid + u * _NW
        h = uu // 128
        b0 = (uu % 128) * 128
        return flat_in.at[pl.ds((h * BATCH + b0) * D_MODEL, 128 * D_MODEL)]

    def dst_slice(u):
        uu = wid + u * _NW
        h = uu // 128
        b0 = (uu % 128) * 128
        return out_hbm.at[h, :, pl.ds(b0, 128)]

    def start_in(u, b):
        @pl.when(u < _C_UNITS)
        def _():
            pltpu.async_copy(src_slice(u), vins[b], isems[b])

    def unit(u2, carry):
        for b in range(2):
            u = u2 * 2 + b
            pltpu.make_async_copy(src_slice(u), vins[b], isems[b]).wait()

            @pl.when(u >= 2)
            def _():
                pltpu.make_async_copy(
                    vtiles[b], dst_slice(u - 2), osems[b]
                ).wait()

            vin = vins[b]
            vtile = vtiles[b]

            @plsc.parallel_loop(0, 256, unroll=16)
            def _(p):
                d = p % D_MODEL
                l = p // D_MODEL
                v = plsc.load_gather(vin, [lanes + (l * 16 * D_MODEL + d)])
                vtile[d, pl.ds(l * 16, 16)] = v

            pltpu.async_copy(vtile, dst_slice(u), osems[b])
            start_in(u + 2, b)
        return carry

    start_in(0, 0)
    start_in(1, 1)
    lax.fori_loop(0, _C_UNITS // 2, unit, 0)
    pltpu.make_async_copy(vtiles[0], dst_slice(_C_UNITS - 2), osems[0]).wait()
    pltpu.make_async_copy(vtiles[1], dst_slice(_C_UNITS - 1), osems[1]).wait()


def kernel(inputs, embedding_matrix):
    tbl_t = jnp.swapaxes(embedding_matrix, 0, 1)
    flat_idx = inputs.reshape(B_FLAT).astype(jnp.int32)
    tail_rows = lax.slice(
        embedding_matrix, (999936, 0), (VOCAB, D_MODEL)
    ).reshape(64 * D_MODEL)
    tbl_lin = _detranspose(tbl_t, tail_rows).reshape(VOCAB, D_MODEL)
    g = _gather_scatter(flat_idx, tbl_lin)
    out_t = _assemble(g.reshape(B_FLAT * D_MODEL))
    return jnp.transpose(out_t, (2, 0, 1))


# speedup vs baseline: 1.7249x; 1.0002x over previous
import functools

import jax
import jax.numpy as jnp
from jax import lax
from jax.experimental import pallas as pl
from jax.experimental.pallas import tpu as pltpu
from jax.experimental.pallas import tpu_sc as plsc

VOCAB = 1000000
D_MODEL = 32
BATCH = 16384
HIST = 50
B_FLAT = BATCH * HIST

_NC = 2
_NS = 16
_NW = _NC * _NS

_mesh = plsc.VectorSubcoreMesh(core_axis_name="c", subcore_axis_name="s")

_A_CH = 512
_A_FULL = 1953
_A_ITERS = 62


@functools.partial(
    pl.kernel,
    mesh=_mesh,
    compiler_params=pltpu.CompilerParams(needs_layout_passes=False),
    out_type=jax.ShapeDtypeStruct((VOCAB * D_MODEL,), jnp.float32),
    scratch_types=[
        tuple(pltpu.VMEM((D_MODEL, _A_CH), jnp.float32) for _ in range(2)),
        tuple(pltpu.VMEM((_A_CH * D_MODEL,), jnp.float32) for _ in range(2)),
        tuple(pltpu.SemaphoreType.DMA for _ in range(2)),
        tuple(pltpu.SemaphoreType.DMA for _ in range(2)),
    ],
)
def _detranspose(tbl_t, tail_rows, out_hbm, vins, vouts, isems, osems):
    wid = lax.axis_index("s") * _NC + lax.axis_index("c")
    rows16 = lax.iota(jnp.int32, 16)

    def col0(k):
        return (wid + k * _NW) * _A_CH

    def start_in(k, b):
        @pl.when(wid + k * _NW < _A_FULL)
        def _():
            pltpu.async_copy(
                tbl_t.at[:, pl.ds(col0(k), _A_CH)], vins[b], isems[b]
            )

    def wait_in(k, b):
        @pl.when(wid + k * _NW < _A_FULL)
        def _():
            pltpu.make_async_copy(
                tbl_t.at[:, pl.ds(col0(k), _A_CH)], vins[b], isems[b]
            ).wait()

    def start_out(k, b):
        @pl.when(wid + k * _NW < _A_FULL)
        def _():
            pltpu.async_copy(
                vouts[b], out_hbm.at[pl.ds(col0(k) * D_MODEL, _A_CH * D_MODEL)],
                osems[b],
            )

    def wait_out(k, b):
        @pl.when(wid + k * _NW < _A_FULL)
        def _():
            pltpu.make_async_copy(
                vouts[b], out_hbm.at[pl.ds(col0(k) * D_MODEL, _A_CH * D_MODEL)],
                osems[b],
            ).wait()

    def compute(k, b):
        @pl.when(wid + k * _NW < _A_FULL)
        def _():
            vin = vins[b]
            vout = vouts[b]

            @plsc.parallel_loop(0, _A_CH, unroll=16)
            def _(il):
                cols = jnp.full((16,), il, jnp.int32)
                vout[pl.ds(il * D_MODEL, 16)] = plsc.load_gather(
                    vin, [rows16, cols]
                )
                vout[pl.ds(il * D_MODEL + 16, 16)] = plsc.load_gather(
                    vin, [rows16 + 16, cols]
                )

    start_in(0, 0)
    start_in(1, 1)

    def chunk_loop(k2, carry):
        for b in range(2):
            k = k2 * 2 + b
            wait_in(k, b)

            @pl.when(k >= 2)
            def _():
                wait_out(k - 2, b)

            compute(k, b)
            start_out(k, b)
            start_in(k + 2, b)
        return carry

    lax.fori_loop(0, _A_ITERS // 2, chunk_loop, 0)
    wait_out(_A_ITERS - 2, 0)
    wait_out(_A_ITERS - 1, 1)

    @pl.when(wid == 0)
    def _():
        pltpu.sync_copy(tail_rows, vouts[0].at[pl.ds(0, 64 * D_MODEL)])
        pltpu.sync_copy(
            vouts[0].at[pl.ds(0, 64 * D_MODEL)],
            out_hbm.at[pl.ds(999936 * D_MODEL, 64 * D_MODEL)],
        )


_B_CH = 1024
_B_PER_W = B_FLAT // _NW
_B_ITERS = _B_PER_W // _B_CH


@functools.partial(
    pl.kernel,
    mesh=_mesh,
    compiler_params=pltpu.CompilerParams(
        use_tc_tiling_on_sc=False, needs_layout_passes=False
    ),
    out_type=jax.ShapeDtypeStruct((B_FLAT, D_MODEL), jnp.float32),
    scratch_types=[
        tuple(pltpu.VMEM((_B_CH,), jnp.int32) for _ in range(2)),
        tuple(pltpu.VMEM((_B_CH, D_MODEL), jnp.float32) for _ in range(2)),
        tuple(pltpu.VMEM((128,), jnp.int32) for _ in range(8)),
        tuple(pltpu.SemaphoreType.DMA for _ in range(2)),
        tuple(pltpu.SemaphoreType.DMA for _ in range(2)),
    ],
)
def _gather_scatter(idx_hbm, tbl_lin, out_hbm, idxvs, rowss, drefs, gsems, ssems):
    wid = lax.axis_index("s") * _NC + lax.axis_index("c")
    base = wid * _B_PER_W
    rows16 = lax.iota(jnp.int32, 16)

    def start_gather(kc, b):
        @pl.when(kc < _B_ITERS)
        def _():
            j0 = base + kc * _B_CH
            pltpu.sync_copy(idx_hbm.at[pl.ds(j0, _B_CH)], idxvs[b])
            pltpu.async_copy(tbl_lin.at[idxvs[b]], rowss[b], gsems[b])

    start_gather(0, 0)
    start_gather(1, 1)

    def chunk(k2, carry):
        for b in range(2):
            kc = k2 * 2 + b

            @pl.when(kc < _B_ITERS)
            def _():
                j0 = base + kc * _B_CH
                rows = rowss[b]
                pltpu.make_async_copy(
                    tbl_lin.at[idxvs[b]], rows, gsems[b]
                ).wait()
                for sub in range(8):
                    dref = drefs[sub]
                    for l in range(8):
                        jv = (
                            jnp.full(
                                (16,), j0 + sub * 128 + l * 16, jnp.int32
                            )
                            + rows16
                        )
                        h = jv % HIST
                        bb = jv // HIST
                        dref[pl.ds(l * 16, 16)] = h * BATCH + bb
                    pltpu.async_copy(
                        rows.at[pl.ds(sub * 128, 128)],
                        out_hbm.at[dref],
                        ssems[b],
                    )
                for sub in range(8):
                    pltpu.make_async_copy(
                        rows.at[pl.ds(sub * 128, 128)],
                        out_hbm.at[drefs[sub]],
                        ssems[b],
                    ).wait()

            start_gather(kc + 2, b)
        return carry

    lax.fori_loop(0, (_B_ITERS + 2) // 2, chunk, 0)


_C_UNITS = (HIST * BATCH) // (128 * _NW)


@functools.partial(
    pl.kernel,
    mesh=_mesh,
    compiler_params=pltpu.CompilerParams(needs_layout_passes=False),
    out_type=jax.ShapeDtypeStruct((HIST, D_MODEL, BATCH), jnp.float32),
    scratch_types=[
        tuple(pltpu.VMEM((128 * D_MODEL,), jnp.float32) for _ in range(2)),
        tuple(pltpu.VMEM((D_MODEL, 128), jnp.float32) for _ in range(2)),
        tuple(pltpu.SemaphoreType.DMA for _ in range(2)),
        tuple(pltpu.SemaphoreType.DMA for _ in range(2)),
    ],
)
def _assemble(flat_in, out_hbm, vins, vtiles, isems, osems):
    wid = lax.axis_index("s") * _NC + lax.axis_index("c")
    lanes = lax.iota(jnp.int32, 16) * D_MODEL

    def src_slice(u):
        uu = wid + u * _NW
        h = uu // 128
        b0 = (uu % 128) * 128
        return flat_in.at[pl.ds((h * BATCH + b0) * D_MODEL, 128 * D_MODEL)]

    def dst_slice(u):
        uu = wid + u * _NW
        h = uu // 128
        b0 = (uu % 128) * 128
        return out_hbm.at[h, :, pl.ds(b0, 128)]

    def start_in(u, b):
        @pl.when(u < _C_UNITS)
        def _():
            pltpu.async_copy(src_slice(u), vins[b], isems[b])

    def unit(u2, carry):
        for b in range(2):
            u = u2 * 2 + b
            pltpu.make_async_copy(src_slice(u), vins[b], isems[b]).wait()

            @pl.when(u >= 2)
            def _():
                pltpu.make_async_copy(
                    vtiles[b], dst_slice(u - 2), osems[b]
                ).wait()

            vin = vins[b]
            vtile = vtiles[b]

            @plsc.parallel_loop(0, 256, unroll=16)
            def _(p):
                d = p % D_MODEL
                l = p // D_MODEL
                v = plsc.load_gather(vin, [lanes + (l * 16 * D_MODEL + d)])
                vtile[d, pl.ds(l * 16, 16)] = v

            pltpu.async_copy(vtile, dst_slice(u), osems[b])
            start_in(u + 2, b)
        return carry

    start_in(0, 0)
    start_in(1, 1)
    lax.fori_loop(0, _C_UNITS // 2, unit, 0)
    pltpu.make_async_copy(vtiles[0], dst_slice(_C_UNITS - 2), osems[0]).wait()
    pltpu.make_async_copy(vtiles[1], dst_slice(_C_UNITS - 1), osems[1]).wait()


def kernel(inputs, embedding_matrix):
    tbl_t = jnp.swapaxes(embedding_matrix, 0, 1)
    flat_idx = inputs.reshape(B_FLAT).astype(jnp.int32)
    tail_rows = lax.slice(
        embedding_matrix, (999936, 0), (VOCAB, D_MODEL)
    ).reshape(64 * D_MODEL)
    tbl_lin = _detranspose(tbl_t, tail_rows).reshape(VOCAB, D_MODEL)
    g = _gather_scatter(flat_idx, tbl_lin)
    out_t = _assemble(g.reshape(B_FLAT * D_MODEL))
    return jnp.transpose(out_t, (2, 0, 1))


# speedup vs baseline: 1.7496x; 1.0143x over previous
import functools

import jax
import jax.numpy as jnp
from jax import lax
from jax.experimental import pallas as pl
from jax.experimental.pallas import tpu as pltpu
from jax.experimental.pallas import tpu_sc as plsc

VOCAB = 1000000
D_MODEL = 32
BATCH = 16384
HIST = 50
B_FLAT = BATCH * HIST

_NC = 2
_NS = 16
_NW = _NC * _NS

_mesh = plsc.VectorSubcoreMesh(core_axis_name="c", subcore_axis_name="s")

_A_CH = 512
_A_FULL = 1953
_A_ITERS = 62


@functools.partial(
    pl.kernel,
    mesh=_mesh,
    compiler_params=pltpu.CompilerParams(needs_layout_passes=False),
    out_type=jax.ShapeDtypeStruct((VOCAB * D_MODEL,), jnp.float32),
    scratch_types=[
        tuple(pltpu.VMEM((D_MODEL, _A_CH), jnp.float32) for _ in range(2)),
        tuple(pltpu.VMEM((_A_CH * D_MODEL,), jnp.float32) for _ in range(2)),
        tuple(pltpu.SemaphoreType.DMA for _ in range(2)),
        tuple(pltpu.SemaphoreType.DMA for _ in range(2)),
    ],
)
def _detranspose(tbl_t, tail_rows, out_hbm, vins, vouts, isems, osems):
    wid = lax.axis_index("s") * _NC + lax.axis_index("c")
    rows16 = lax.iota(jnp.int32, 16)

    def col0(k):
        return (wid + k * _NW) * _A_CH

    def start_in(k, b):
        @pl.when(wid + k * _NW < _A_FULL)
        def _():
            pltpu.async_copy(
                tbl_t.at[:, pl.ds(col0(k), _A_CH)], vins[b], isems[b]
            )

    def wait_in(k, b):
        @pl.when(wid + k * _NW < _A_FULL)
        def _():
            pltpu.make_async_copy(
                tbl_t.at[:, pl.ds(col0(k), _A_CH)], vins[b], isems[b]
            ).wait()

    def start_out(k, b):
        @pl.when(wid + k * _NW < _A_FULL)
        def _():
            pltpu.async_copy(
                vouts[b], out_hbm.at[pl.ds(col0(k) * D_MODEL, _A_CH * D_MODEL)],
                osems[b],
            )

    def wait_out(k, b):
        @pl.when(wid + k * _NW < _A_FULL)
        def _():
            pltpu.make_async_copy(
                vouts[b], out_hbm.at[pl.ds(col0(k) * D_MODEL, _A_CH * D_MODEL)],
                osems[b],
            ).wait()

    def compute(k, b):
        @pl.when(wid + k * _NW < _A_FULL)
        def _():
            vin = vins[b]
            vout = vouts[b]

            @plsc.parallel_loop(0, _A_CH, unroll=16)
            def _(il):
                cols = jnp.full((16,), il, jnp.int32)
                vout[pl.ds(il * D_MODEL, 16)] = plsc.load_gather(
                    vin, [rows16, cols]
                )
                vout[pl.ds(il * D_MODEL + 16, 16)] = plsc.load_gather(
                    vin, [rows16 + 16, cols]
                )

    start_in(0, 0)
    start_in(1, 1)

    def chunk_loop(k2, carry):
        for b in range(2):
            k = k2 * 2 + b
            wait_in(k, b)

            @pl.when(k >= 2)
            def _():
                wait_out(k - 2, b)

            compute(k, b)
            start_out(k, b)
            start_in(k + 2, b)
        return carry

    lax.fori_loop(0, _A_ITERS // 2, chunk_loop, 0)
    wait_out(_A_ITERS - 2, 0)
    wait_out(_A_ITERS - 1, 1)

    @pl.when(wid == 0)
    def _():
        pltpu.sync_copy(tail_rows, vouts[0].at[pl.ds(0, 64 * D_MODEL)])
        pltpu.sync_copy(
            vouts[0].at[pl.ds(0, 64 * D_MODEL)],
            out_hbm.at[pl.ds(999936 * D_MODEL, 64 * D_MODEL)],
        )


_B_CH = 1024
_B_PER_W = B_FLAT // _NW
_B_ITERS = _B_PER_W // _B_CH


@functools.partial(
    pl.kernel,
    mesh=_mesh,
    compiler_params=pltpu.CompilerParams(
        use_tc_tiling_on_sc=False, needs_layout_passes=False
    ),
    out_type=jax.ShapeDtypeStruct((B_FLAT, D_MODEL), jnp.float32),
    scratch_types=[
        tuple(pltpu.VMEM((_B_CH,), jnp.int32) for _ in range(2)),
        tuple(pltpu.VMEM((_B_CH, D_MODEL), jnp.float32) for _ in range(2)),
        tuple(pltpu.VMEM((128,), jnp.int32) for _ in range(8)),
        tuple(pltpu.SemaphoreType.DMA for _ in range(2)),
        tuple(pltpu.SemaphoreType.DMA for _ in range(2)),
    ],
)
def _gather_scatter(idx_hbm, tbl_lin, out_hbm, idxvs, rowss, drefs, gsems, ssems):
    wid = lax.axis_index("s") * _NC + lax.axis_index("c")
    base = wid * _B_PER_W
    rows16 = lax.iota(jnp.int32, 16)

    def start_gather(kc, b):
        @pl.when(kc < _B_ITERS)
        def _():
            j0 = base + kc * _B_CH
            pltpu.sync_copy(idx_hbm.at[pl.ds(j0, _B_CH)], idxvs[b])
            pltpu.async_copy(tbl_lin.at[idxvs[b]], rowss[b], gsems[b])

    start_gather(0, 0)
    start_gather(1, 1)

    def chunk(k2, carry):
        for b in range(2):
            kc = k2 * 2 + b

            @pl.when(kc < _B_ITERS)
            def _():
                j0 = base + kc * _B_CH
                rows = rowss[b]
                pltpu.make_async_copy(
                    tbl_lin.at[idxvs[b]], rows, gsems[b]
                ).wait()
                for sub in range(8):
                    dref = drefs[sub]
                    for l in range(8):
                        jv = (
                            jnp.full(
                                (16,), j0 + sub * 128 + l * 16, jnp.int32
                            )
                            + rows16
                        )
                        h = jv % HIST
                        bb = jv // HIST
                        dref[pl.ds(l * 16, 16)] = h * BATCH + bb
                    pltpu.async_copy(
                        rows.at[pl.ds(sub * 128, 128)],
                        out_hbm.at[dref],
                        ssems[b],
                    )
                for sub in range(8):
                    pltpu.make_async_copy(
                        rows.at[pl.ds(sub * 128, 128)],
                        out_hbm.at[drefs[sub]],
                        ssems[b],
                    ).wait()

            start_gather(kc + 2, b)
        return carry

    lax.fori_loop(0, (_B_ITERS + 2) // 2, chunk, 0)


_C_UNITS = (HIST * BATCH) // (128 * _NW)


@functools.partial(
    pl.kernel,
    mesh=_mesh,
    compiler_params=pltpu.CompilerParams(needs_layout_passes=False),
    out_type=jax.ShapeDtypeStruct((HIST, D_MODEL, BATCH), jnp.float32),
    scratch_types=[
        tuple(pltpu.VMEM((128 * D_MODEL,), jnp.float32) for _ in range(2)),
        tuple(pltpu.VMEM((D_MODEL, 128), jnp.float32) for _ in range(2)),
        tuple(pltpu.SemaphoreType.DMA for _ in range(2)),
        tuple(pltpu.SemaphoreType.DMA for _ in range(2)),
    ],
)
def _assemble(flat_in, out_hbm, vins, vtiles, isems, osems):
    wid = lax.axis_index("s") * _NC + lax.axis_index("c")
    lanes = lax.iota(jnp.int32, 16) * D_MODEL

    def src_slice(u):
        uu = wid + u * _NW
        h = uu // 128
        b0 = (uu % 128) * 128
        return flat_in.at[pl.ds((h * BATCH + b0) * D_MODEL, 128 * D_MODEL)]

    def dst_slice(u):
        uu = wid + u * _NW
        h = uu // 128
        b0 = (uu % 128) * 128
        return out_hbm.at[h, :, pl.ds(b0, 128)]

    def start_in(u, b):
        @pl.when(u < _C_UNITS)
        def _():
            pltpu.async_copy(src_slice(u), vins[b], isems[b])

    def unit(u2, carry):
        for b in range(2):
            u = u2 * 2 + b
            pltpu.make_async_copy(src_slice(u), vins[b], isems[b]).wait()

            @pl.when(u >= 2)
            def _():
                pltpu.make_async_copy(
                    vtiles[b], dst_slice(u - 2), osems[b]
                ).wait()

            vin = vins[b]
            vtile = vtiles[b]

            @plsc.parallel_loop(0, D_MODEL, unroll=4)
            def _(d):
                base = lanes + d
                for l in range(8):
                    v = plsc.load_gather(vin, [base + l * 16 * D_MODEL])
                    vtile[d, pl.ds(l * 16, 16)] = v

            pltpu.async_copy(vtile, dst_slice(u), osems[b])
            start_in(u + 2, b)
        return carry

    start_in(0, 0)
    start_in(1, 1)
    lax.fori_loop(0, _C_UNITS // 2, unit, 0)
    pltpu.make_async_copy(vtiles[0], dst_slice(_C_UNITS - 2), osems[0]).wait()
    pltpu.make_async_copy(vtiles[1], dst_slice(_C_UNITS - 1), osems[1]).wait()


def kernel(inputs, embedding_matrix):
    tbl_t = jnp.swapaxes(embedding_matrix, 0, 1)
    flat_idx = inputs.reshape(B_FLAT).astype(jnp.int32)
    tail_rows = lax.slice(
        embedding_matrix, (999936, 0), (VOCAB, D_MODEL)
    ).reshape(64 * D_MODEL)
    tbl_lin = _detranspose(tbl_t, tail_rows).reshape(VOCAB, D_MODEL)
    g = _gather_scatter(flat_idx, tbl_lin)
    out_t = _assemble(g.reshape(B_FLAT * D_MODEL))
    return jnp.transpose(out_t, (2, 0, 1))
